# baseline (speedup 1.0000x reference)
import functools

import jax
import jax.numpy as jnp
from jax import lax
from jax.experimental import pallas as pl
from jax.experimental.pallas import tpu as pltpu
from jax.experimental.pallas import tpu_sc as plsc

N_EMBED = 128
BATCH = 4096
HIST = 50
NC = 2
NS = 16
NW = NC * NS
NCH = 2
CB = BATCH // NCH
BPW = CB // NW
K = 4
NSB = BPW // K
NBUF = 4
HP = 56

_mesh = plsc.VectorSubcoreMesh(core_axis_name="c", subcore_axis_name="s")


@functools.partial(
    pl.kernel,
    out_type=jax.ShapeDtypeStruct((CB, HIST, N_EMBED), jnp.float32),
    mesh=_mesh,
    scratch_types=[
        pltpu.VMEM((BPW, HIST), jnp.int32),
        [pltpu.VMEM((K, HIST, N_EMBED), jnp.float32) for _ in range(NBUF)],
        [pltpu.SemaphoreType.DMA for _ in range(NBUF)],
        [pltpu.SemaphoreType.DMA for _ in range(NBUF)],
    ],
)
def _gather_chunk(idx_hbm, table_hbm, out_hbm, idx_v, bufs, gsems, psems):
    wid = lax.axis_index("s") * NC + lax.axis_index("c")
    b0 = wid * BPW
    pltpu.sync_copy(idx_hbm.at[pl.ds(b0, BPW)], idx_v)

    def fire(s, r):
        for i in range(K):
            pltpu.async_copy(table_hbm.at[idx_v.at[s * K + i]], bufs[r].at[i],
                             gsems[r])

    def drain(s, r):
        for i in range(K):
            pltpu.make_async_copy(table_hbm.at[idx_v.at[s * K + i]],
                                  bufs[r].at[i], gsems[r]).wait()

    def fire_put(s, r):
        return pltpu.async_copy(bufs[r], out_hbm.at[pl.ds(b0 + s * K, K)],
                                psems[r])

    def wait_put(s, r):
        pltpu.make_async_copy(bufs[r], out_hbm.at[pl.ds(b0 + s * K, K)],
                              psems[r]).wait()

    for r in range(NBUF - 1):
        fire(r, r)

    def outer(t, carry):
        for r in range(NBUF):
            s = t * NBUF + r
            drain(s, r)
            fire_put(s, r)

            @pl.when(s >= 1)
            def _():
                wait_put(s - 1, (r + NBUF - 1) % NBUF)

            @pl.when(s + NBUF - 1 < NSB)
            def _():
                fire(s + NBUF - 1, (r + NBUF - 1) % NBUF)

        return carry

    lax.fori_loop(0, NSB // NBUF, outer, 0)
    wait_put(NSB - 1, (NSB - 1) % NBUF)


def kernel(input, embedding):
    idx = input.astype(jnp.int32)
    chunks = [_gather_chunk(idx[c * CB:(c + 1) * CB], embedding)
              for c in range(NCH)]
    return jnp.concatenate(chunks, axis=0)


# speedup vs baseline: 1.5998x; 1.5998x over previous
import functools

import jax
import jax.numpy as jnp
from jax import lax
from jax.experimental import pallas as pl
from jax.experimental.pallas import tpu as pltpu
from jax.experimental.pallas import tpu_sc as plsc

N_EMBED = 128
BATCH = 4096
HIST = 50
NC = 2
NS = 16
NW = NC * NS
NCH = 1
CB = BATCH // NCH
BPW = CB // NW
K = 4
NSB = BPW // K
NBUF = 4
HP = 56

_mesh = plsc.VectorSubcoreMesh(core_axis_name="c", subcore_axis_name="s")


@functools.partial(
    pl.kernel,
    out_type=jax.ShapeDtypeStruct((CB, HIST, N_EMBED), jnp.float32),
    mesh=_mesh,
    scratch_types=[
        pltpu.VMEM((BPW, HIST), jnp.int32),
        [pltpu.VMEM((K, HIST, N_EMBED), jnp.float32) for _ in range(NBUF)],
        [pltpu.SemaphoreType.DMA for _ in range(NBUF)],
        [pltpu.SemaphoreType.DMA for _ in range(NBUF)],
    ],
)
def _gather_chunk(idx_hbm, table_hbm, out_hbm, idx_v, bufs, gsems, psems):
    wid = lax.axis_index("s") * NC + lax.axis_index("c")
    b0 = wid * BPW
    pltpu.sync_copy(idx_hbm.at[pl.ds(b0, BPW)], idx_v)

    def fire(s, r):
        for i in range(K):
            pltpu.async_copy(table_hbm.at[idx_v.at[s * K + i]], bufs[r].at[i],
                             gsems[r])

    def drain(s, r):
        for i in range(K):
            pltpu.make_async_copy(table_hbm.at[idx_v.at[s * K + i]],
                                  bufs[r].at[i], gsems[r]).wait()

    def fire_put(s, r):
        return pltpu.async_copy(bufs[r], out_hbm.at[pl.ds(b0 + s * K, K)],
                                psems[r])

    def wait_put(s, r):
        pltpu.make_async_copy(bufs[r], out_hbm.at[pl.ds(b0 + s * K, K)],
                              psems[r]).wait()

    for r in range(NBUF - 1):
        fire(r, r)

    def outer(t, carry):
        for r in range(NBUF):
            s = t * NBUF + r
            drain(s, r)
            fire_put(s, r)

            @pl.when(s >= 1)
            def _():
                wait_put(s - 1, (r + NBUF - 1) % NBUF)

            @pl.when(s + NBUF - 1 < NSB)
            def _():
                fire(s + NBUF - 1, (r + NBUF - 1) % NBUF)

        return carry

    lax.fori_loop(0, NSB // NBUF, outer, 0)
    wait_put(NSB - 1, (NSB - 1) % NBUF)


def kernel(input, embedding):
    return _gather_chunk(input.astype(jnp.int32), embedding)


# speedup vs baseline: 1.6054x; 1.0035x over previous
import functools

import jax
import jax.numpy as jnp
from jax import lax
from jax.experimental import pallas as pl
from jax.experimental.pallas import tpu as pltpu
from jax.experimental.pallas import tpu_sc as plsc

N_EMBED = 128
BATCH = 4096
HIST = 50
NC = 2
NS = 16
NW = NC * NS
NCH = 1
CB = BATCH // NCH
BPW = CB // NW
K = 2
NSB = BPW // K
NBUF = 8
HP = 56

_mesh = plsc.VectorSubcoreMesh(core_axis_name="c", subcore_axis_name="s")


@functools.partial(
    pl.kernel,
    out_type=jax.ShapeDtypeStruct((CB, HIST, N_EMBED), jnp.float32),
    mesh=_mesh,
    scratch_types=[
        pltpu.VMEM((BPW, HIST), jnp.int32),
        [pltpu.VMEM((K, HIST, N_EMBED), jnp.float32) for _ in range(NBUF)],
        [pltpu.SemaphoreType.DMA for _ in range(NBUF)],
        [pltpu.SemaphoreType.DMA for _ in range(NBUF)],
    ],
)
def _gather_chunk(idx_hbm, table_hbm, out_hbm, idx_v, bufs, gsems, psems):
    wid = lax.axis_index("s") * NC + lax.axis_index("c")
    b0 = wid * BPW
    pltpu.sync_copy(idx_hbm.at[pl.ds(b0, BPW)], idx_v)

    def fire(s, r):
        for i in range(K):
            pltpu.async_copy(table_hbm.at[idx_v.at[s * K + i]], bufs[r].at[i],
                             gsems[r])

    def drain(s, r):
        for i in range(K):
            pltpu.make_async_copy(table_hbm.at[idx_v.at[s * K + i]],
                                  bufs[r].at[i], gsems[r]).wait()

    def fire_put(s, r):
        return pltpu.async_copy(bufs[r], out_hbm.at[pl.ds(b0 + s * K, K)],
                                psems[r])

    def wait_put(s, r):
        pltpu.make_async_copy(bufs[r], out_hbm.at[pl.ds(b0 + s * K, K)],
                              psems[r]).wait()

    for r in range(NBUF - 1):
        fire(r, r)

    def outer(t, carry):
        for r in range(NBUF):
            s = t * NBUF + r
            drain(s, r)
            fire_put(s, r)

            @pl.when(s >= 1)
            def _():
                wait_put(s - 1, (r + NBUF - 1) % NBUF)

            @pl.when(s + NBUF - 1 < NSB)
            def _():
                fire(s + NBUF - 1, (r + NBUF - 1) % NBUF)

        return carry

    lax.fori_loop(0, NSB // NBUF, outer, 0)
    wait_put(NSB - 1, (NSB - 1) % NBUF)


def kernel(input, embedding):
    return _gather_chunk(input.astype(jnp.int32), embedding)
